# baseline (speedup 1.0000x reference)
import jax
import jax.numpy as jnp
from jax.experimental import pallas as pl

B_ORT, B_REG, B_CLS = 0.2, 2.0, 1.0
_B = 4
_P = 40000
_SUB, _LANE = 8, 5000
_CLS_TOTAL = float(_B * 2 * _P)


def _loss_kernel(x_ref, t_ref, rg_ref, rta_ref, rtb_ref, out_ref):
    cls_sum = 0.0
    sl1_sum = 0.0
    ce_sum = 0.0
    npos = 0.0
    rows7 = jax.lax.broadcasted_iota(jnp.int32, (7, _SUB, _LANE), 0)
    for b in range(_B):
        x = x_ref[b]
        t = t_ref[b]
        p = jax.nn.sigmoid(x)
        pt = jnp.where(t == 1.0, p, 1.0 - p)
        at = jnp.where(t == 1.0, 1000.0, 1.0)
        q = 1.0 - pt
        w = at * q * q
        bce = jnp.maximum(x, 0.0) - x * t + jnp.log1p(jnp.exp(-jnp.abs(x)))
        cls_sum += jnp.sum(w * bce)

        rt_ref = rta_ref if b < 2 else rtb_ref
        bb = b % 2
        for a in range(2):
            mask = (rt_ref[bb, 9 * a] == 1.0).astype(jnp.float32)
            npos += jnp.sum(mask)
            s = rg_ref[b, 9 * a:9 * a + 7]
            if a == 0:
                s = jnp.where(rows7 == 6, jnp.tanh(s), s)
            d = s - rt_ref[bb, 9 * a + 1:9 * a + 8]
            ad = jnp.abs(d)
            sl1 = jnp.where(ad < 1.0, 0.5 * d * d, ad - 0.5)
            sl1_sum += jnp.sum(sl1 * mask[None])
            z0 = rg_ref[b, 9 * a + 7]
            z1 = rg_ref[b, 9 * a + 8]
            tc = rt_ref[bb, 9 * a + 8]
            diff = jnp.where(tc == 1.0, z0 - z1, z1 - z0)
            ce = jnp.maximum(diff, 0.0) + jnp.log1p(jnp.exp(-jnp.abs(diff)))
            ce_sum += jnp.sum(ce * mask)

    cls_loss = cls_sum / _CLS_TOTAL
    reg_loss = sl1_sum / (npos * 7.0)
    ort_loss = ce_sum / npos
    loss = B_CLS * cls_loss + B_ORT * ort_loss + B_REG * reg_loss
    out_ref[...] = jnp.full((1, 1), loss, dtype=jnp.float32)


def kernel(cls_tensor, reg_tensor, cls_targets, reg_targets):
    x = cls_tensor.reshape(_B, 2, _SUB, _LANE)
    t = cls_targets.transpose(0, 3, 1, 2).reshape(_B, 2, _SUB, _LANE)
    rg = reg_tensor.reshape(_B, 18, _SUB, _LANE)
    rt4 = reg_targets.reshape(_B, _P, 2, 9)
    rta = (rt4[:2].transpose(0, 2, 3, 1).reshape(2, 18, _SUB, _LANE))
    rtb = (rt4[2:].transpose(0, 2, 3, 1).reshape(2, 18, _SUB, _LANE))

    out = pl.pallas_call(
        _loss_kernel,
        out_shape=jax.ShapeDtypeStruct((1, 1), jnp.float32),
    )(x, t, rg, rta, rtb)
    return out[0, 0]
